# baseline (speedup 1.0000x reference)
import functools

import jax
import jax.numpy as jnp
from jax import lax
from jax.experimental import pallas as pl
from jax.experimental.pallas import tpu as pltpu
from jax.experimental.pallas import tpu_sc as plsc

N = 10000
E = 320000
H = 32
NC, NS, LN = 2, 16, 16
NW = NC * NS
NPT = 640
NPAD = NS * NPT
EPT = E // NW
EPS = E // NS
CHR = 1000
CHS = 2000
SPT = NPAD // NW
TBL = N * N + 8
SHORTCUT = 14


@functools.cache
def _mesh():
    return plsc.VectorSubcoreMesh(core_axis_name="c", subcore_axis_name="s",
                                  num_cores=NC, num_subcores=NS)


def _params():
    return pltpu.CompilerParams(use_tc_tiling_on_sc=False,
                                needs_layout_passes=False)


def _fill(ref, size, value, dtype):
    @pl.loop(0, size // LN)
    def _(i):
        ref[pl.ds(i * LN, LN)] = jnp.full((LN,), value, dtype)


def _zero_rows(zrow_v):
    @pl.loop(0, NPT)
    def _(i):
        zrow_v[i, pl.ds(0, LN)] = jnp.zeros((LN,), jnp.float32)
        zrow_v[i, pl.ds(LN, LN)] = jnp.zeros((LN,), jnp.float32)


@functools.cache
def _deg_kernel():
    @functools.partial(
        pl.kernel,
        out_type=jax.ShapeDtypeStruct((NC, NPAD), jnp.float32),
        mesh=_mesh(),
        scratch_types=[
            pltpu.VMEM((EPT,), jnp.int32),
            pltpu.VMEM((EPT,), jnp.float32),
            pltpu.VMEM((NPT,), jnp.float32),
            pltpu.VMEM_SHARED((NPAD,), jnp.float32),
        ],
        compiler_params=_params(),
    )
    def deg(dst_hbm, out_hbm, idx_v, ones_v, z_v, acc_sh):
        c = lax.axis_index("c")
        s = lax.axis_index("s")
        _fill(z_v, NPT, 0.0, jnp.float32)
        pltpu.sync_copy(z_v, acc_sh.at[pl.ds(s * NPT, NPT)])
        _fill(ones_v, EPT, 1.0, jnp.float32)
        plsc.subcore_barrier()
        base = (c * NS + s) * EPT
        pltpu.sync_copy(dst_hbm.at[pl.ds(base, EPT)], idx_v)
        pltpu.sync_copy(ones_v, acc_sh.at[idx_v], add=True)
        plsc.subcore_barrier()

        @pl.when(s == 0)
        def _():
            pltpu.sync_copy(acc_sh, out_hbm.at[c])

    return deg


@functools.cache
def _agg_kernel():
    @functools.partial(
        pl.kernel,
        out_type=jax.ShapeDtypeStruct((NC, NPAD, H), jnp.float32),
        mesh=_mesh(),
        scratch_types=[
            pltpu.VMEM((CHR,), jnp.int32),
            pltpu.VMEM((CHR,), jnp.int32),
            pltpu.VMEM((CHR, H), jnp.float32),
            pltpu.VMEM((NPT, H), jnp.float32),
            pltpu.VMEM_SHARED((N, H), jnp.float32),
            pltpu.VMEM_SHARED((NPAD, H), jnp.float32),
            pltpu.SemaphoreType.DMA,
        ],
        compiler_params=_params(),
    )
    def agg(rows_hbm, src_hbm, dst_hbm, out_hbm,
            idx_v, tgt_v, rows_v, zrow_v, tab_sh, acc_sh, sem):
        c = lax.axis_index("c")
        s = lax.axis_index("s")
        _zero_rows(zrow_v)
        pltpu.sync_copy(zrow_v, acc_sh.at[pl.ds(s * NPT, NPT)])

        @pl.when(s == 0)
        def _():
            pltpu.sync_copy(rows_hbm, tab_sh)

        plsc.subcore_barrier()
        base = (c * NS + s) * EPT

        @pl.loop(0, EPT // CHR)
        def _(k):
            off = base + k * CHR
            pltpu.sync_copy(src_hbm.at[pl.ds(off, CHR)], idx_v)
            pltpu.sync_copy(dst_hbm.at[pl.ds(off, CHR)], tgt_v)
            pltpu.async_copy(tab_sh.at[idx_v], rows_v, sem).wait()
            pltpu.sync_copy(rows_v, acc_sh.at[tgt_v], add=True)

        plsc.subcore_barrier()

        @pl.when(s == 0)
        def _():
            pltpu.sync_copy(acc_sh, out_hbm.at[c])

    return agg


@functools.cache
def _agg_w_kernel(with_singles):
    scratch = [
        pltpu.VMEM((CHR,), jnp.int32),
        pltpu.VMEM((CHR,), jnp.int32),
        pltpu.VMEM((CHR,), jnp.float32),
        pltpu.VMEM((CHR, H), jnp.float32),
        pltpu.VMEM((NPT, H), jnp.float32),
        pltpu.VMEM_SHARED((N, H), jnp.float32),
        pltpu.VMEM_SHARED((NPAD, H), jnp.float32),
        pltpu.SemaphoreType.DMA,
    ]
    if with_singles:
        scratch = scratch + [
            pltpu.VMEM((SPT,), jnp.int32),
            pltpu.VMEM((SPT, H), jnp.float32),
        ]

    def body(*refs):
        if with_singles:
            (rows_hbm, idx_hbm, tgt_hbm, w_hbm, sing_rows_hbm, sing_tgt_hbm,
             out_hbm, idx_v, tgt_v, w_v, rows_v, zrow_v, tab_sh, acc_sh, sem,
             stgt_v, srows_v) = refs
        else:
            (rows_hbm, idx_hbm, tgt_hbm, w_hbm, out_hbm,
             idx_v, tgt_v, w_v, rows_v, zrow_v, tab_sh, acc_sh, sem) = refs
        c = lax.axis_index("c")
        s = lax.axis_index("s")
        _zero_rows(zrow_v)
        pltpu.sync_copy(zrow_v, acc_sh.at[pl.ds(s * NPT, NPT)])

        @pl.when(s == 0)
        def _():
            pltpu.sync_copy(rows_hbm, tab_sh)

        plsc.subcore_barrier()
        base = (c * NS + s) * EPT

        @pl.loop(0, EPT // CHR)
        def _(k):
            off = base + k * CHR
            pltpu.sync_copy(idx_hbm.at[pl.ds(off, CHR)], idx_v)
            pltpu.sync_copy(tgt_hbm.at[pl.ds(off, CHR)], tgt_v)
            pltpu.sync_copy(w_hbm.at[pl.ds(off, CHR)], w_v)
            pltpu.async_copy(tab_sh.at[idx_v], rows_v, sem).wait()

            @pl.loop(0, CHR, unroll=4)
            def _(e):
                eidx = jnp.full((LN,), e, jnp.int32)
                wb = plsc.load_gather(w_v, [eidx])
                rows_v[e, pl.ds(0, LN)] = rows_v[e, pl.ds(0, LN)] * wb
                rows_v[e, pl.ds(LN, LN)] = rows_v[e, pl.ds(LN, LN)] * wb

            pltpu.sync_copy(rows_v, acc_sh.at[tgt_v], add=True)

        if with_singles:
            soff = (c * NS + s) * SPT
            pltpu.sync_copy(sing_rows_hbm.at[pl.ds(soff, SPT)], srows_v)
            pltpu.sync_copy(sing_tgt_hbm.at[pl.ds(soff, SPT)], stgt_v)
            pltpu.sync_copy(srows_v, acc_sh.at[stgt_v], add=True)

        plsc.subcore_barrier()

        @pl.when(s == 0)
        def _():
            pltpu.sync_copy(acc_sh, out_hbm.at[c])

    return pl.kernel(
        body,
        out_type=jax.ShapeDtypeStruct((NC, NPAD, H), jnp.float32),
        mesh=_mesh(),
        scratch_types=scratch,
        compiler_params=_params(),
    )


@functools.cache
def _score_kernel():
    @functools.partial(
        pl.kernel,
        out_type=(
            jax.ShapeDtypeStruct((E,), jnp.float32),
            jax.ShapeDtypeStruct((E,), jnp.int32),
            jax.ShapeDtypeStruct((NC, NPAD), jnp.float32),
        ),
        mesh=_mesh(),
        scratch_types=[
            pltpu.VMEM((N,), jnp.float32),
            pltpu.VMEM((N,), jnp.float32),
            pltpu.VMEM((CHS,), jnp.int32),
            pltpu.VMEM((CHS,), jnp.int32),
            pltpu.VMEM((CHS,), jnp.float32),
            pltpu.VMEM((CHS,), jnp.int32),
            pltpu.VMEM((CHS,), jnp.float32),
            pltpu.VMEM((NPT,), jnp.float32),
            pltpu.VMEM_SHARED((NPAD,), jnp.float32),
        ],
        compiler_params=_params(),
    )
    def score(src_hbm, dst_hbm, p_hbm, q_hbm, sm_hbm, tk_hbm, he_hbm,
              p_v, q_v, srcc_v, dstc_v, sm_v, tk_v, tkf_v, z_v, he_sh):
        c = lax.axis_index("c")
        s = lax.axis_index("s")
        _fill(z_v, NPT, 0.0, jnp.float32)
        pltpu.sync_copy(z_v, he_sh.at[pl.ds(s * NPT, NPT)])
        pltpu.sync_copy(p_hbm, p_v)
        pltpu.sync_copy(q_hbm, q_v)
        plsc.subcore_barrier()
        base = (c * NS + s) * EPT

        @pl.loop(0, EPT // CHS)
        def _(k):
            off = base + k * CHS
            pltpu.sync_copy(src_hbm.at[pl.ds(off, CHS)], srcc_v)
            pltpu.sync_copy(dst_hbm.at[pl.ds(off, CHS)], dstc_v)

            @pl.loop(0, CHS // LN)
            def _(j):
                sa = srcc_v[pl.ds(j * LN, LN)]
                sb = dstc_v[pl.ds(j * LN, LN)]
                ps = plsc.load_gather(p_v, [sa])
                qd = plsc.load_gather(q_v, [sb])
                t16 = ps + qd
                sc = 1.0 / (1.0 + jnp.exp(-t16))
                mk = sa != sb
                sm_v[pl.ds(j * LN, LN)] = jnp.where(mk, sc, 0.0)
                tk = mk & (t16 > 0.0)
                tk_v[pl.ds(j * LN, LN)] = jnp.where(tk, 1, 0).astype(jnp.int32)
                tkf_v[pl.ds(j * LN, LN)] = jnp.where(tk, 1.0, 0.0)

            pltpu.sync_copy(sm_v, sm_hbm.at[pl.ds(off, CHS)])
            pltpu.sync_copy(tk_v, tk_hbm.at[pl.ds(off, CHS)])
            pltpu.sync_copy(tkf_v, he_sh.at[srcc_v], add=True)
            pltpu.sync_copy(tkf_v, he_sh.at[dstc_v], add=True)

        plsc.subcore_barrier()

        @pl.when(s == 0)
        def _():
            pltpu.sync_copy(he_sh, he_hbm.at[c])

    return score


@functools.cache
def _cc_kernel():
    @functools.partial(
        pl.kernel,
        out_type=(
            jax.ShapeDtypeStruct((NPAD,), jnp.int32),
            jax.ShapeDtypeStruct((E,), jnp.int32),
            jax.ShapeDtypeStruct((E,), jnp.int32),
        ),
        mesh=_mesh(),
        scratch_types=[
            pltpu.VMEM((NPAD,), jnp.int32),
            pltpu.VMEM((NPT,), jnp.int32),
            pltpu.VMEM((NPT,), jnp.int32),
            pltpu.VMEM((CHS,), jnp.int32),
            pltpu.VMEM((CHS,), jnp.int32),
            pltpu.VMEM((CHS,), jnp.int32),
            pltpu.VMEM((CHS,), jnp.int32),
            pltpu.VMEM((CHS,), jnp.int32),
            pltpu.VMEM((LN,), jnp.int32),
            pltpu.VMEM((NS, LN), jnp.int32),
            pltpu.VMEM_SHARED((NS, NPAD), jnp.int32),
            pltpu.VMEM_SHARED((NPAD,), jnp.int32),
            pltpu.VMEM_SHARED((NS, LN), jnp.int32),
        ],
        compiler_params=_params(),
    )
    def cc(src_hbm, dst_hbm, tk_hbm, lab_hbm, cs_hbm, cd_hbm,
           L_v, j_v, t_v, srcc_v, dstc_v, tkc_v, cs_v, cd_v,
           cnt_v, flg_v, lcop_sh, canon_sh, flag_sh):
        c = lax.axis_index("c")
        s = lax.axis_index("s")

        @pl.when(c == 0)
        def _():
            iota = lax.iota(jnp.int32, LN)

            @pl.loop(0, NPAD // LN)
            def _(i):
                L_v[pl.ds(i * LN, LN)] = iota + i * LN

            pltpu.sync_copy(L_v.at[pl.ds(s * NPT, NPT)],
                            canon_sh.at[pl.ds(s * NPT, NPT)])
            plsc.subcore_barrier()

            def _round(state):
                del state

                def hook_chunk(k, mism):
                    off = s * EPS + k * CHS
                    pltpu.sync_copy(src_hbm.at[pl.ds(off, CHS)], srcc_v)
                    pltpu.sync_copy(dst_hbm.at[pl.ds(off, CHS)], dstc_v)
                    pltpu.sync_copy(tk_hbm.at[pl.ds(off, CHS)], tkc_v)

                    def hook16(j, mism):
                        sa = srcc_v[pl.ds(j * LN, LN)]
                        sb = dstc_v[pl.ds(j * LN, LN)]
                        tk = tkc_v[pl.ds(j * LN, LN)]
                        a = plsc.load_gather(L_v, [sa])
                        b = plsc.load_gather(L_v, [sb])
                        hi = jnp.maximum(a, b)
                        lo = jnp.minimum(a, b)
                        act = (tk > 0) & (hi != lo)
                        g = plsc.load_gather(L_v, [hi])
                        newv = jnp.minimum(g, lo)
                        plsc.store_scatter(L_v, [hi], newv, mask=act)
                        return mism + jnp.where(act, 1, 0).astype(jnp.int32)

                    return lax.fori_loop(0, CHS // LN, hook16, mism)

                mism = lax.fori_loop(0, EPS // CHS, hook_chunk,
                                     jnp.zeros((LN,), jnp.int32))
                pltpu.sync_copy(L_v, lcop_sh.at[s])
                cnt_v[pl.ds(0, LN)] = mism
                pltpu.sync_copy(cnt_v, flag_sh.at[s])
                plsc.subcore_barrier()

                pltpu.sync_copy(lcop_sh.at[0, pl.ds(s * NPT, NPT)], j_v)
                for t in range(1, NS):
                    pltpu.sync_copy(lcop_sh.at[t, pl.ds(s * NPT, NPT)], t_v)

                    @pl.loop(0, NPT // LN)
                    def _(i):
                        j_v[pl.ds(i * LN, LN)] = jnp.minimum(
                            j_v[pl.ds(i * LN, LN)], t_v[pl.ds(i * LN, LN)])

                pltpu.sync_copy(flag_sh, flg_v)
                tot16 = jnp.zeros((LN,), jnp.int32)
                for t in range(NS):
                    tot16 = tot16 + flg_v[t, pl.ds(0, LN)]
                total = jnp.sum(tot16)

                pltpu.sync_copy(j_v, canon_sh.at[pl.ds(s * NPT, NPT)])
                plsc.subcore_barrier()
                pltpu.sync_copy(canon_sh, L_v)

                @pl.loop(0, NPT // LN)
                def _(i):
                    v = j_v[pl.ds(i * LN, LN)]
                    for _ in range(SHORTCUT):
                        v = plsc.load_gather(L_v, [v])
                    j_v[pl.ds(i * LN, LN)] = v

                pltpu.sync_copy(j_v, canon_sh.at[pl.ds(s * NPT, NPT)])
                plsc.subcore_barrier()
                pltpu.sync_copy(canon_sh, L_v)
                return total

            lax.while_loop(lambda t: t > 0, _round, jnp.int32(1))

            @pl.when(s == 0)
            def _():
                pltpu.sync_copy(L_v, lab_hbm)

            @pl.loop(0, EPS // CHS)
            def _(k):
                off = s * EPS + k * CHS
                pltpu.sync_copy(src_hbm.at[pl.ds(off, CHS)], srcc_v)
                pltpu.sync_copy(dst_hbm.at[pl.ds(off, CHS)], dstc_v)

                @pl.loop(0, CHS // LN)
                def _(j):
                    sa = srcc_v[pl.ds(j * LN, LN)]
                    sb = dstc_v[pl.ds(j * LN, LN)]
                    cs_v[pl.ds(j * LN, LN)] = plsc.load_gather(L_v, [sa])
                    cd_v[pl.ds(j * LN, LN)] = plsc.load_gather(L_v, [sb])

                pltpu.sync_copy(cs_v, cs_hbm.at[pl.ds(off, CHS)])
                pltpu.sync_copy(cd_v, cd_hbm.at[pl.ds(off, CHS)])

    return cc


@functools.cache
def _dedup_kernel():
    @functools.partial(
        pl.kernel,
        out_type=(
            jax.ShapeDtypeStruct((TBL,), jnp.int32),
            jax.ShapeDtypeStruct((E,), jnp.float32),
            jax.ShapeDtypeStruct((NPAD,), jnp.float32),
        ),
        mesh=_mesh(),
        scratch_types=[
            pltpu.VMEM((CHS,), jnp.int32),
            pltpu.VMEM((CHS,), jnp.int32),
            pltpu.VMEM((CHS,), jnp.int32),
            pltpu.VMEM((CHS,), jnp.int32),
            pltpu.VMEM((CHS,), jnp.int32),
            pltpu.VMEM((CHS,), jnp.int32),
            pltpu.VMEM((CHS,), jnp.int32),
            pltpu.VMEM((CHS,), jnp.float32),
            pltpu.VMEM((NPT,), jnp.float32),
            pltpu.VMEM_SHARED((NPAD,), jnp.float32),
            pltpu.SemaphoreType.DMA,
        ],
        compiler_params=_params(),
    )
    def dedup(src_hbm, dst_hbm, cs_hbm, cd_hbm, tbl_hbm, rep_hbm, deg2_hbm,
              srcc_v, dstc_v, cs_v, cd_v, code_v, eid_v, g_v, rep_v,
              z_v, deg_sh, sem):
        c = lax.axis_index("c")
        s = lax.axis_index("s")

        @pl.when(c == 0)
        def _():
            iota = lax.iota(jnp.int32, LN)
            _fill(z_v, NPT, 0.0, jnp.float32)
            pltpu.sync_copy(z_v, deg_sh.at[pl.ds(s * NPT, NPT)])

            def load_and_code(k):
                off = s * EPS + k * CHS
                pltpu.sync_copy(src_hbm.at[pl.ds(off, CHS)], srcc_v)
                pltpu.sync_copy(dst_hbm.at[pl.ds(off, CHS)], dstc_v)
                pltpu.sync_copy(cs_hbm.at[pl.ds(off, CHS)], cs_v)
                pltpu.sync_copy(cd_hbm.at[pl.ds(off, CHS)], cd_v)

                @pl.loop(0, CHS // LN)
                def _(j):
                    sa = srcc_v[pl.ds(j * LN, LN)]
                    sb = dstc_v[pl.ds(j * LN, LN)]
                    csk = cs_v[pl.ds(j * LN, LN)]
                    cdk = cd_v[pl.ds(j * LN, LN)]
                    mmk = (sa != sb) & (csk != cdk)
                    code = csk * N + cdk
                    code_v[pl.ds(j * LN, LN)] = jnp.where(mmk, code, N * N)
                    eid_v[pl.ds(j * LN, LN)] = iota + (off + j * LN)

                return off

            @pl.loop(0, EPS // CHS)
            def _(k):
                load_and_code(k)
                pltpu.sync_copy(eid_v, tbl_hbm.at[code_v])

            plsc.subcore_barrier()

            @pl.loop(0, EPS // CHS)
            def _(k):
                off = load_and_code(k)
                pltpu.async_copy(tbl_hbm.at[code_v], g_v, sem).wait()

                @pl.loop(0, CHS // LN)
                def _(j):
                    gk = g_v[pl.ds(j * LN, LN)]
                    ek = eid_v[pl.ds(j * LN, LN)]
                    ck = code_v[pl.ds(j * LN, LN)]
                    rep = (gk == ek) & (ck < N * N)
                    rep_v[pl.ds(j * LN, LN)] = jnp.where(rep, 1.0, 0.0)

                pltpu.sync_copy(rep_v, rep_hbm.at[pl.ds(off, CHS)])
                pltpu.sync_copy(rep_v, deg_sh.at[cd_v], add=True)

            plsc.subcore_barrier()

            @pl.when(s == 0)
            def _():
                pltpu.sync_copy(deg_sh, deg2_hbm)

    return dedup


def _mm_kernel(x_ref, w_ref, o_ref):
    o_ref[...] = jnp.dot(x_ref[...], w_ref[...],
                         preferred_element_type=jnp.float32)


def _matmul(x, w):
    return pl.pallas_call(
        _mm_kernel,
        out_shape=jax.ShapeDtypeStruct((x.shape[0], w.shape[1]), jnp.float32),
    )(x, w)


def kernel(x, edge_index, edge_weight, batch, W1, b1, Wp, bp, W3, b3, Wf, bf):
    src = edge_index[:, 0]
    dst = edge_index[:, 1]

    h = _matmul(x, W1)
    deg1 = 1.0 + _deg_kernel()(dst).sum(axis=0)[:N]
    dinv1 = lax.rsqrt(deg1)
    hs = dinv1[:, None] * h
    acc1 = _agg_kernel()(hs, src, dst).sum(axis=0)[:N]
    x1 = jax.nn.relu(dinv1[:, None] * acc1 + dinv1[:, None] ** 2 * h + b1)

    w2col = jnp.concatenate([Wp[:H], Wp[H:]], axis=1)
    pq = _matmul(x1, w2col)
    p = pq[:, 0]
    q = pq[:, 1] + bp[0]
    sm, tk, he_parts = _score_kernel()(src, dst, p, q)
    he = he_parts.sum(axis=0)[:N]

    take = tk > 0

    def cc_round(state):
        L, _ = state
        a = L[src]
        b = L[dst]
        hi = jnp.maximum(a, b)
        lo = jnp.minimum(a, b)
        hi = jnp.where(take & (hi != lo), hi, N)
        L2 = L.at[hi].min(lo, mode="drop")
        L2 = lax.fori_loop(0, 14, lambda _, v: v[v], L2)
        return L2, jnp.any(L2 != L)

    labels, _ = lax.while_loop(lambda st: st[1], cc_round,
                               (jnp.arange(N, dtype=jnp.int32),
                                jnp.bool_(True)))
    lab_pad = jnp.concatenate(
        [labels, jnp.arange(N, NPAD, dtype=jnp.int32)])
    csrc = labels[src]
    cdst = labels[dst]

    single = (he == 0.0)
    sing_rows = jnp.where(single[:, None], x1, 0.0)
    sing_rows_pad = jnp.concatenate(
        [sing_rows, jnp.zeros((NPAD - N, H), jnp.float32)], axis=0)
    xc = _agg_w_kernel(True)(x1, src, cdst, sm, sing_rows_pad,
                             lab_pad).sum(axis=0)[:N]

    _tbl, rep, deg2e = _dedup_kernel()(src, dst, csrc, cdst)
    deg2 = 1.0 + deg2e[:N]
    dinv2 = lax.rsqrt(deg2)

    hc = _matmul(xc, W3)
    w2 = dinv2[:, None] * hc
    acc2 = _agg_w_kernel(False)(w2, csrc, cdst, rep).sum(axis=0)[:N]
    x2 = jax.nn.relu(dinv2[:, None] * acc2 + dinv2[:, None] ** 2 * hc + b3)

    is_root = (labels == jnp.arange(N, dtype=jnp.int32)).astype(jnp.float32)
    K = is_root.sum()
    pooled = (x2 * is_root[:, None]).sum(axis=0) / K
    out = jax.nn.sigmoid(pooled @ Wf + bf)
    return out.reshape(-1)


# speedup vs baseline: 25.7061x; 25.7061x over previous
import functools

import jax
import jax.numpy as jnp
from jax import lax
from jax.experimental import pallas as pl
from jax.experimental.pallas import tpu as pltpu
from jax.experimental.pallas import tpu_sc as plsc

N = 10000
E = 320000
H = 32
NC, NS, LN = 2, 16, 16
NW = NC * NS
NPT = 640
NPAD = NS * NPT
EPT = E // NW
EPS = E // NS
CHR = 1000
CHS = 2000
SPT = NPAD // NW
TBL = N * N + 8
SHORTCUT = 14


@functools.cache
def _mesh():
    return plsc.VectorSubcoreMesh(core_axis_name="c", subcore_axis_name="s",
                                  num_cores=NC, num_subcores=NS)


def _params():
    return pltpu.CompilerParams(use_tc_tiling_on_sc=False,
                                needs_layout_passes=False)


def _fill(ref, size, value, dtype):
    @pl.loop(0, size // LN)
    def _(i):
        ref[pl.ds(i * LN, LN)] = jnp.full((LN,), value, dtype)


def _zero_rows(zrow_v):
    @pl.loop(0, NPT)
    def _(i):
        zrow_v[i, pl.ds(0, LN)] = jnp.zeros((LN,), jnp.float32)
        zrow_v[i, pl.ds(LN, LN)] = jnp.zeros((LN,), jnp.float32)


@functools.cache
def _deg_kernel():
    @functools.partial(
        pl.kernel,
        out_type=jax.ShapeDtypeStruct((NC, NPAD), jnp.float32),
        mesh=_mesh(),
        scratch_types=[
            pltpu.VMEM((EPT,), jnp.int32),
            pltpu.VMEM((EPT,), jnp.float32),
            pltpu.VMEM((NPT,), jnp.float32),
            pltpu.VMEM_SHARED((NPAD,), jnp.float32),
        ],
        compiler_params=_params(),
    )
    def deg(dst_hbm, out_hbm, idx_v, ones_v, z_v, acc_sh):
        c = lax.axis_index("c")
        s = lax.axis_index("s")
        _fill(z_v, NPT, 0.0, jnp.float32)
        pltpu.sync_copy(z_v, acc_sh.at[pl.ds(s * NPT, NPT)])
        _fill(ones_v, EPT, 1.0, jnp.float32)
        plsc.subcore_barrier()
        base = (c * NS + s) * EPT
        pltpu.sync_copy(dst_hbm.at[pl.ds(base, EPT)], idx_v)
        pltpu.sync_copy(ones_v, acc_sh.at[idx_v], add=True)
        plsc.subcore_barrier()

        @pl.when(s == 0)
        def _():
            pltpu.sync_copy(acc_sh, out_hbm.at[c])

    return deg


@functools.cache
def _agg_kernel():
    @functools.partial(
        pl.kernel,
        out_type=jax.ShapeDtypeStruct((NC, NPAD, H), jnp.float32),
        mesh=_mesh(),
        scratch_types=[
            pltpu.VMEM((CHR,), jnp.int32),
            pltpu.VMEM((CHR,), jnp.int32),
            pltpu.VMEM((CHR, H), jnp.float32),
            pltpu.VMEM((NPT, H), jnp.float32),
            pltpu.VMEM_SHARED((N, H), jnp.float32),
            pltpu.VMEM_SHARED((NPAD, H), jnp.float32),
            pltpu.SemaphoreType.DMA,
        ],
        compiler_params=_params(),
    )
    def agg(rows_hbm, src_hbm, dst_hbm, out_hbm,
            idx_v, tgt_v, rows_v, zrow_v, tab_sh, acc_sh, sem):
        c = lax.axis_index("c")
        s = lax.axis_index("s")
        _zero_rows(zrow_v)
        pltpu.sync_copy(zrow_v, acc_sh.at[pl.ds(s * NPT, NPT)])

        @pl.when(s == 0)
        def _():
            pltpu.sync_copy(rows_hbm, tab_sh)

        plsc.subcore_barrier()
        base = (c * NS + s) * EPT

        @pl.loop(0, EPT // CHR)
        def _(k):
            off = base + k * CHR
            pltpu.sync_copy(src_hbm.at[pl.ds(off, CHR)], idx_v)
            pltpu.sync_copy(dst_hbm.at[pl.ds(off, CHR)], tgt_v)
            pltpu.async_copy(tab_sh.at[idx_v], rows_v, sem).wait()
            pltpu.sync_copy(rows_v, acc_sh.at[tgt_v], add=True)

        plsc.subcore_barrier()

        @pl.when(s == 0)
        def _():
            pltpu.sync_copy(acc_sh, out_hbm.at[c])

    return agg


@functools.cache
def _agg_w_kernel(with_singles):
    scratch = [
        pltpu.VMEM((CHR,), jnp.int32),
        pltpu.VMEM((CHR,), jnp.int32),
        pltpu.VMEM((CHR,), jnp.float32),
        pltpu.VMEM((CHR, H), jnp.float32),
        pltpu.VMEM((NPT, H), jnp.float32),
        pltpu.VMEM_SHARED((N, H), jnp.float32),
        pltpu.VMEM_SHARED((NPAD, H), jnp.float32),
        pltpu.SemaphoreType.DMA,
    ]
    if with_singles:
        scratch = scratch + [
            pltpu.VMEM((SPT,), jnp.int32),
            pltpu.VMEM((SPT, H), jnp.float32),
        ]

    def body(*refs):
        if with_singles:
            (rows_hbm, idx_hbm, tgt_hbm, w_hbm, sing_rows_hbm, sing_tgt_hbm,
             out_hbm, idx_v, tgt_v, w_v, rows_v, zrow_v, tab_sh, acc_sh, sem,
             stgt_v, srows_v) = refs
        else:
            (rows_hbm, idx_hbm, tgt_hbm, w_hbm, out_hbm,
             idx_v, tgt_v, w_v, rows_v, zrow_v, tab_sh, acc_sh, sem) = refs
        c = lax.axis_index("c")
        s = lax.axis_index("s")
        _zero_rows(zrow_v)
        pltpu.sync_copy(zrow_v, acc_sh.at[pl.ds(s * NPT, NPT)])

        @pl.when(s == 0)
        def _():
            pltpu.sync_copy(rows_hbm, tab_sh)

        plsc.subcore_barrier()
        base = (c * NS + s) * EPT

        @pl.loop(0, EPT // CHR)
        def _(k):
            off = base + k * CHR
            pltpu.sync_copy(idx_hbm.at[pl.ds(off, CHR)], idx_v)
            pltpu.sync_copy(tgt_hbm.at[pl.ds(off, CHR)], tgt_v)
            pltpu.sync_copy(w_hbm.at[pl.ds(off, CHR)], w_v)
            pltpu.async_copy(tab_sh.at[idx_v], rows_v, sem).wait()

            @pl.loop(0, CHR, unroll=4)
            def _(e):
                eidx = jnp.full((LN,), e, jnp.int32)
                wb = plsc.load_gather(w_v, [eidx])
                rows_v[e, pl.ds(0, LN)] = rows_v[e, pl.ds(0, LN)] * wb
                rows_v[e, pl.ds(LN, LN)] = rows_v[e, pl.ds(LN, LN)] * wb

            pltpu.sync_copy(rows_v, acc_sh.at[tgt_v], add=True)

        if with_singles:
            soff = (c * NS + s) * SPT
            pltpu.sync_copy(sing_rows_hbm.at[pl.ds(soff, SPT)], srows_v)
            pltpu.sync_copy(sing_tgt_hbm.at[pl.ds(soff, SPT)], stgt_v)
            pltpu.sync_copy(srows_v, acc_sh.at[stgt_v], add=True)

        plsc.subcore_barrier()

        @pl.when(s == 0)
        def _():
            pltpu.sync_copy(acc_sh, out_hbm.at[c])

    return pl.kernel(
        body,
        out_type=jax.ShapeDtypeStruct((NC, NPAD, H), jnp.float32),
        mesh=_mesh(),
        scratch_types=scratch,
        compiler_params=_params(),
    )


@functools.cache
def _score_kernel():
    @functools.partial(
        pl.kernel,
        out_type=(
            jax.ShapeDtypeStruct((E,), jnp.float32),
            jax.ShapeDtypeStruct((E,), jnp.int32),
            jax.ShapeDtypeStruct((NC, NPAD), jnp.float32),
        ),
        mesh=_mesh(),
        scratch_types=[
            pltpu.VMEM((N,), jnp.float32),
            pltpu.VMEM((N,), jnp.float32),
            pltpu.VMEM((CHS,), jnp.int32),
            pltpu.VMEM((CHS,), jnp.int32),
            pltpu.VMEM((CHS,), jnp.float32),
            pltpu.VMEM((CHS,), jnp.int32),
            pltpu.VMEM((CHS,), jnp.float32),
            pltpu.VMEM((NPT,), jnp.float32),
            pltpu.VMEM_SHARED((NPAD,), jnp.float32),
        ],
        compiler_params=_params(),
    )
    def score(src_hbm, dst_hbm, p_hbm, q_hbm, sm_hbm, tk_hbm, he_hbm,
              p_v, q_v, srcc_v, dstc_v, sm_v, tk_v, tkf_v, z_v, he_sh):
        c = lax.axis_index("c")
        s = lax.axis_index("s")
        _fill(z_v, NPT, 0.0, jnp.float32)
        pltpu.sync_copy(z_v, he_sh.at[pl.ds(s * NPT, NPT)])
        pltpu.sync_copy(p_hbm, p_v)
        pltpu.sync_copy(q_hbm, q_v)
        plsc.subcore_barrier()
        base = (c * NS + s) * EPT

        @pl.loop(0, EPT // CHS)
        def _(k):
            off = base + k * CHS
            pltpu.sync_copy(src_hbm.at[pl.ds(off, CHS)], srcc_v)
            pltpu.sync_copy(dst_hbm.at[pl.ds(off, CHS)], dstc_v)

            @pl.loop(0, CHS // LN)
            def _(j):
                sa = srcc_v[pl.ds(j * LN, LN)]
                sb = dstc_v[pl.ds(j * LN, LN)]
                ps = plsc.load_gather(p_v, [sa])
                qd = plsc.load_gather(q_v, [sb])
                t16 = ps + qd
                sc = 1.0 / (1.0 + jnp.exp(-t16))
                mk = sa != sb
                sm_v[pl.ds(j * LN, LN)] = jnp.where(mk, sc, 0.0)
                tk = mk & (t16 > 0.0)
                tk_v[pl.ds(j * LN, LN)] = jnp.where(tk, 1, 0).astype(jnp.int32)
                tkf_v[pl.ds(j * LN, LN)] = jnp.where(tk, 1.0, 0.0)

            pltpu.sync_copy(sm_v, sm_hbm.at[pl.ds(off, CHS)])
            pltpu.sync_copy(tk_v, tk_hbm.at[pl.ds(off, CHS)])
            pltpu.sync_copy(tkf_v, he_sh.at[srcc_v], add=True)
            pltpu.sync_copy(tkf_v, he_sh.at[dstc_v], add=True)

        plsc.subcore_barrier()

        @pl.when(s == 0)
        def _():
            pltpu.sync_copy(he_sh, he_hbm.at[c])

    return score


@functools.cache
def _cc_kernel():
    @functools.partial(
        pl.kernel,
        out_type=(
            jax.ShapeDtypeStruct((NPAD,), jnp.int32),
            jax.ShapeDtypeStruct((E,), jnp.int32),
            jax.ShapeDtypeStruct((E,), jnp.int32),
        ),
        mesh=_mesh(),
        scratch_types=[
            pltpu.VMEM((NPAD,), jnp.int32),
            pltpu.VMEM((NPT,), jnp.int32),
            pltpu.VMEM((NPT,), jnp.int32),
            pltpu.VMEM((CHS,), jnp.int32),
            pltpu.VMEM((CHS,), jnp.int32),
            pltpu.VMEM((CHS,), jnp.int32),
            pltpu.VMEM((CHS,), jnp.int32),
            pltpu.VMEM((CHS,), jnp.int32),
            pltpu.VMEM((LN,), jnp.int32),
            pltpu.VMEM((NS, LN), jnp.int32),
            pltpu.VMEM_SHARED((NS, NPAD), jnp.int32),
            pltpu.VMEM_SHARED((NPAD,), jnp.int32),
            pltpu.VMEM_SHARED((NS, LN), jnp.int32),
        ],
        compiler_params=_params(),
    )
    def cc(src_hbm, dst_hbm, tk_hbm, lab_hbm, cs_hbm, cd_hbm,
           L_v, j_v, t_v, srcc_v, dstc_v, tkc_v, cs_v, cd_v,
           cnt_v, flg_v, lcop_sh, canon_sh, flag_sh):
        c = lax.axis_index("c")
        s = lax.axis_index("s")

        @pl.when(c == 0)
        def _():
            iota = lax.iota(jnp.int32, LN)

            @pl.loop(0, NPAD // LN)
            def _(i):
                L_v[pl.ds(i * LN, LN)] = iota + i * LN

            pltpu.sync_copy(L_v.at[pl.ds(s * NPT, NPT)],
                            canon_sh.at[pl.ds(s * NPT, NPT)])
            plsc.subcore_barrier()

            def _round(state):
                del state

                def hook_chunk(k, mism):
                    off = s * EPS + k * CHS
                    pltpu.sync_copy(src_hbm.at[pl.ds(off, CHS)], srcc_v)
                    pltpu.sync_copy(dst_hbm.at[pl.ds(off, CHS)], dstc_v)
                    pltpu.sync_copy(tk_hbm.at[pl.ds(off, CHS)], tkc_v)

                    def hook16(j, mism):
                        sa = srcc_v[pl.ds(j * LN, LN)]
                        sb = dstc_v[pl.ds(j * LN, LN)]
                        tk = tkc_v[pl.ds(j * LN, LN)]
                        a = plsc.load_gather(L_v, [sa])
                        b = plsc.load_gather(L_v, [sb])
                        hi = jnp.maximum(a, b)
                        lo = jnp.minimum(a, b)
                        act = (tk > 0) & (hi != lo)
                        g = plsc.load_gather(L_v, [hi])
                        newv = jnp.minimum(g, lo)
                        plsc.store_scatter(L_v, [hi], newv, mask=act)
                        return mism + jnp.where(act, 1, 0).astype(jnp.int32)

                    return lax.fori_loop(0, CHS // LN, hook16, mism)

                mism = lax.fori_loop(0, EPS // CHS, hook_chunk,
                                     jnp.zeros((LN,), jnp.int32))
                pltpu.sync_copy(L_v, lcop_sh.at[s])
                cnt_v[pl.ds(0, LN)] = mism
                pltpu.sync_copy(cnt_v, flag_sh.at[s])
                plsc.subcore_barrier()

                pltpu.sync_copy(lcop_sh.at[0, pl.ds(s * NPT, NPT)], j_v)
                for t in range(1, NS):
                    pltpu.sync_copy(lcop_sh.at[t, pl.ds(s * NPT, NPT)], t_v)

                    @pl.loop(0, NPT // LN)
                    def _(i):
                        j_v[pl.ds(i * LN, LN)] = jnp.minimum(
                            j_v[pl.ds(i * LN, LN)], t_v[pl.ds(i * LN, LN)])

                pltpu.sync_copy(flag_sh, flg_v)
                tot16 = jnp.zeros((LN,), jnp.int32)
                for t in range(NS):
                    tot16 = tot16 + flg_v[t, pl.ds(0, LN)]
                total = jnp.sum(tot16)

                pltpu.sync_copy(j_v, canon_sh.at[pl.ds(s * NPT, NPT)])
                plsc.subcore_barrier()
                pltpu.sync_copy(canon_sh, L_v)

                @pl.loop(0, NPT // LN)
                def _(i):
                    v = j_v[pl.ds(i * LN, LN)]
                    for _ in range(SHORTCUT):
                        v = plsc.load_gather(L_v, [v])
                    j_v[pl.ds(i * LN, LN)] = v

                pltpu.sync_copy(j_v, canon_sh.at[pl.ds(s * NPT, NPT)])
                plsc.subcore_barrier()
                pltpu.sync_copy(canon_sh, L_v)
                return total

            lax.while_loop(lambda t: t > 0, _round, jnp.int32(1))

            @pl.when(s == 0)
            def _():
                pltpu.sync_copy(L_v, lab_hbm)

            @pl.loop(0, EPS // CHS)
            def _(k):
                off = s * EPS + k * CHS
                pltpu.sync_copy(src_hbm.at[pl.ds(off, CHS)], srcc_v)
                pltpu.sync_copy(dst_hbm.at[pl.ds(off, CHS)], dstc_v)

                @pl.loop(0, CHS // LN)
                def _(j):
                    sa = srcc_v[pl.ds(j * LN, LN)]
                    sb = dstc_v[pl.ds(j * LN, LN)]
                    cs_v[pl.ds(j * LN, LN)] = plsc.load_gather(L_v, [sa])
                    cd_v[pl.ds(j * LN, LN)] = plsc.load_gather(L_v, [sb])

                pltpu.sync_copy(cs_v, cs_hbm.at[pl.ds(off, CHS)])
                pltpu.sync_copy(cd_v, cd_hbm.at[pl.ds(off, CHS)])

    return cc


@functools.cache
def _dedup_kernel():
    @functools.partial(
        pl.kernel,
        out_type=(
            jax.ShapeDtypeStruct((TBL,), jnp.int32),
            jax.ShapeDtypeStruct((E,), jnp.float32),
            jax.ShapeDtypeStruct((NPAD,), jnp.float32),
        ),
        mesh=_mesh(),
        scratch_types=[
            pltpu.VMEM((CHS,), jnp.int32),
            pltpu.VMEM((CHS,), jnp.int32),
            pltpu.VMEM((CHS,), jnp.int32),
            pltpu.VMEM((CHS,), jnp.int32),
            pltpu.VMEM((CHS,), jnp.int32),
            pltpu.VMEM((CHS,), jnp.int32),
            pltpu.VMEM((CHS,), jnp.int32),
            pltpu.VMEM((CHS,), jnp.float32),
            pltpu.VMEM((NPT,), jnp.float32),
            pltpu.VMEM_SHARED((NPAD,), jnp.float32),
            pltpu.SemaphoreType.DMA,
        ],
        compiler_params=_params(),
    )
    def dedup(src_hbm, dst_hbm, cs_hbm, cd_hbm, tbl_hbm, rep_hbm, deg2_hbm,
              srcc_v, dstc_v, cs_v, cd_v, code_v, eid_v, g_v, rep_v,
              z_v, deg_sh, sem):
        c = lax.axis_index("c")
        s = lax.axis_index("s")

        @pl.when(c == 0)
        def _():
            iota = lax.iota(jnp.int32, LN)
            _fill(z_v, NPT, 0.0, jnp.float32)
            pltpu.sync_copy(z_v, deg_sh.at[pl.ds(s * NPT, NPT)])

            def load_and_code(k):
                off = s * EPS + k * CHS
                pltpu.sync_copy(src_hbm.at[pl.ds(off, CHS)], srcc_v)
                pltpu.sync_copy(dst_hbm.at[pl.ds(off, CHS)], dstc_v)
                pltpu.sync_copy(cs_hbm.at[pl.ds(off, CHS)], cs_v)
                pltpu.sync_copy(cd_hbm.at[pl.ds(off, CHS)], cd_v)

                @pl.loop(0, CHS // LN)
                def _(j):
                    sa = srcc_v[pl.ds(j * LN, LN)]
                    sb = dstc_v[pl.ds(j * LN, LN)]
                    csk = cs_v[pl.ds(j * LN, LN)]
                    cdk = cd_v[pl.ds(j * LN, LN)]
                    mmk = (sa != sb) & (csk != cdk)
                    code = csk * N + cdk
                    code_v[pl.ds(j * LN, LN)] = jnp.where(mmk, code, N * N)
                    eid_v[pl.ds(j * LN, LN)] = iota + (off + j * LN)

                return off

            @pl.loop(0, EPS // CHS)
            def _(k):
                load_and_code(k)
                pltpu.sync_copy(eid_v, tbl_hbm.at[code_v])

            plsc.subcore_barrier()

            @pl.loop(0, EPS // CHS)
            def _(k):
                off = load_and_code(k)
                pltpu.async_copy(tbl_hbm.at[code_v], g_v, sem).wait()

                @pl.loop(0, CHS // LN)
                def _(j):
                    gk = g_v[pl.ds(j * LN, LN)]
                    ek = eid_v[pl.ds(j * LN, LN)]
                    ck = code_v[pl.ds(j * LN, LN)]
                    rep = (gk == ek) & (ck < N * N)
                    rep_v[pl.ds(j * LN, LN)] = jnp.where(rep, 1.0, 0.0)

                pltpu.sync_copy(rep_v, rep_hbm.at[pl.ds(off, CHS)])
                pltpu.sync_copy(rep_v, deg_sh.at[cd_v], add=True)

            plsc.subcore_barrier()

            @pl.when(s == 0)
            def _():
                pltpu.sync_copy(deg_sh, deg2_hbm)

    return dedup


def _mm_kernel(x_ref, w_ref, o_ref):
    o_ref[...] = jnp.dot(x_ref[...], w_ref[...],
                         preferred_element_type=jnp.float32)


def _matmul(x, w):
    return pl.pallas_call(
        _mm_kernel,
        out_shape=jax.ShapeDtypeStruct((x.shape[0], w.shape[1]), jnp.float32),
    )(x, w)


def kernel(x, edge_index, edge_weight, batch, W1, b1, Wp, bp, W3, b3, Wf, bf):
    src = edge_index[:, 0]
    dst = edge_index[:, 1]

    h = _matmul(x, W1)
    deg1 = 1.0 + _deg_kernel()(dst).sum(axis=0)[:N]
    dinv1 = lax.rsqrt(deg1)
    hs = dinv1[:, None] * h
    acc1 = _agg_kernel()(hs, src, dst).sum(axis=0)[:N]
    x1 = jax.nn.relu(dinv1[:, None] * acc1 + dinv1[:, None] ** 2 * h + b1)

    w2col = jnp.concatenate([Wp[:H], Wp[H:]], axis=1)
    pq = _matmul(x1, w2col)
    p = pq[:, 0]
    q = pq[:, 1] + bp[0]
    sm, tk, he_parts = _score_kernel()(src, dst, p, q)
    he = he_parts.sum(axis=0)[:N]

    lab_pad, csrc, cdst = _cc_kernel()(src, dst, tk)
    labels = lab_pad[:N]

    single = (he == 0.0)
    sing_rows = jnp.where(single[:, None], x1, 0.0)
    sing_rows_pad = jnp.concatenate(
        [sing_rows, jnp.zeros((NPAD - N, H), jnp.float32)], axis=0)
    xc = _agg_w_kernel(True)(x1, src, cdst, sm, sing_rows_pad,
                             lab_pad).sum(axis=0)[:N]

    mmk = (src != dst) & (csrc != cdst)
    big = N * N
    code = jnp.sort(jnp.where(mmk, csrc * N + cdst, big))
    first = jnp.concatenate([jnp.ones((1,), bool), code[1:] != code[:-1]])
    ev = first & (code < big)
    wuniq = ev.astype(jnp.float32)
    nsrc = jnp.where(ev, code // N, 0).astype(jnp.int32)
    ndst = jnp.where(ev, code % N, 0).astype(jnp.int32)
    deg2 = jnp.ones((N,), jnp.float32).at[ndst].add(wuniq)
    dinv2 = lax.rsqrt(deg2)

    hc = _matmul(xc, W3)
    w2 = dinv2[:, None] * hc
    acc2 = _agg_w_kernel(False)(w2, nsrc, ndst, wuniq).sum(axis=0)[:N]
    x2 = jax.nn.relu(dinv2[:, None] * acc2 + dinv2[:, None] ** 2 * hc + b3)

    is_root = (labels == jnp.arange(N, dtype=jnp.int32)).astype(jnp.float32)
    K = is_root.sum()
    pooled = (x2 * is_root[:, None]).sum(axis=0) / K
    out = jax.nn.sigmoid(pooled @ Wf + bf)
    return out.reshape(-1)
